# Initial kernel scaffold; baseline (speedup 1.0000x reference)
import jax
import jax.numpy as jnp
from jax.experimental import pallas as pl


def kernel(DNA_x, tf_x, W):
    raise NotImplementedError("write your pallas kernel here")


# speedup vs baseline: 1.7522x; 1.7522x over previous
import functools

import jax
import jax.numpy as jnp
from jax import lax
from jax.experimental import pallas as pl
from jax.experimental.pallas import tpu as pltpu
from jax.experimental.pallas import tpu_sc as plsc

_NC = 2
_NS = 16
_NW = _NC * _NS

_B = 16384
_L = 39
_D = 32
_N = _B * _L
_IW = 128
_BPW = _N // _NW
_IDX_ROWS = _BPW // _IW
_GPC = 12
_CHUNK = _GPC * _IW
_NCHUNK = _BPW // _CHUNK


@functools.partial(
    pl.kernel,
    out_type=jax.ShapeDtypeStruct((_N, _D), jnp.float32),
    mesh=plsc.VectorSubcoreMesh(
        core_axis_name="c", subcore_axis_name="s",
        num_cores=_NC, num_subcores=_NS,
    ),
    scratch_types=[
        pltpu.VMEM((_IDX_ROWS, _IW), jnp.int32),
        pltpu.VMEM((_CHUNK, _D), jnp.float32),
        pltpu.SemaphoreType.DMA,
    ],
    compiler_params=pltpu.CompilerParams(use_tc_tiling_on_sc=False),
)
def _emb_gather(table_hbm, idx_hbm, out_hbm, idx_v, rows_v, sem):
    wid = lax.axis_index("s") * _NC + lax.axis_index("c")
    pltpu.sync_copy(idx_hbm.at[wid], idx_v)
    base = wid * _BPW

    def step(i, carry):
        descs = []
        for j in range(_GPC):
            descs.append(pltpu.async_copy(
                table_hbm.at[idx_v.at[i * _GPC + j]],
                rows_v.at[pl.ds(j * _IW, _IW)],
                sem,
            ))
        for d in descs:
            d.wait()
        pltpu.sync_copy(rows_v, out_hbm.at[pl.ds(base + i * _CHUNK, _CHUNK)])
        return carry

    lax.fori_loop(0, _NCHUNK, step, 0)


def kernel(DNA_x, tf_x, W):
    idx = tf_x.reshape(-1).astype(jnp.int32).reshape(_NW, _IDX_ROWS, _IW)
    emb = _emb_gather(W, idx)
    return (DNA_x, emb.reshape(_B, _L, _D))


# speedup vs baseline: 1.7692x; 1.0097x over previous
import functools

import jax
import jax.numpy as jnp
from jax import lax
from jax.experimental import pallas as pl
from jax.experimental.pallas import tpu as pltpu
from jax.experimental.pallas import tpu_sc as plsc

_NC = 2
_NS = 16
_NW = _NC * _NS

_B = 16384
_L = 39
_D = 32
_N = _B * _L
_IW = 128
_BPW = _N // _NW
_IDX_ROWS = _BPW // _IW
_GPC = 6
_CHUNK = _GPC * _IW
_NCHUNK = _BPW // _CHUNK
_NPAIR = _NCHUNK // 2


@functools.partial(
    pl.kernel,
    out_type=jax.ShapeDtypeStruct((_N, _D), jnp.float32),
    mesh=plsc.VectorSubcoreMesh(
        core_axis_name="c", subcore_axis_name="s",
        num_cores=_NC, num_subcores=_NS,
    ),
    scratch_types=[
        pltpu.VMEM((_IDX_ROWS, _IW), jnp.int32),
        pltpu.VMEM((_CHUNK, _D), jnp.float32),
        pltpu.VMEM((_CHUNK, _D), jnp.float32),
        pltpu.SemaphoreType.DMA,
        pltpu.SemaphoreType.DMA,
        pltpu.SemaphoreType.DMA,
        pltpu.SemaphoreType.DMA,
    ],
    compiler_params=pltpu.CompilerParams(use_tc_tiling_on_sc=False),
)
def _emb_gather(table_hbm, idx_hbm, out_hbm, idx_v, rows0, rows1,
                sem_g0, sem_g1, sem_s0, sem_s1):
    wid = lax.axis_index("s") * _NC + lax.axis_index("c")
    pltpu.sync_copy(idx_hbm.at[wid], idx_v)
    base = wid * _BPW

    def fire_gathers(i, buf, sem):
        for j in range(_GPC):
            pltpu.async_copy(
                table_hbm.at[idx_v.at[i * _GPC + j]],
                buf.at[pl.ds(j * _IW, _IW)], sem)

    def drain_gathers(i, buf, sem):
        for j in range(_GPC):
            pltpu.make_async_copy(
                table_hbm.at[idx_v.at[i * _GPC + j]],
                buf.at[pl.ds(j * _IW, _IW)], sem).wait()

    def scatter_fire(i, buf, sem):
        pltpu.async_copy(buf, out_hbm.at[pl.ds(base + i * _CHUNK, _CHUNK)], sem)

    def scatter_wait(i, buf, sem):
        pltpu.make_async_copy(
            buf, out_hbm.at[pl.ds(base + i * _CHUNK, _CHUNK)], sem).wait()

    fire_gathers(0, rows0, sem_g0)

    def step(k, carry):
        i0 = 2 * k
        i1 = i0 + 1

        @pl.when(k > 0)
        def _():
            scatter_wait(i1 - 2, rows1, sem_s1)

        fire_gathers(i1, rows1, sem_g1)
        drain_gathers(i0, rows0, sem_g0)
        scatter_fire(i0, rows0, sem_s0)
        scatter_wait(i0, rows0, sem_s0)

        @pl.when(k < _NPAIR - 1)
        def _():
            fire_gathers(i0 + 2, rows0, sem_g0)

        drain_gathers(i1, rows1, sem_g1)
        scatter_fire(i1, rows1, sem_s1)
        return carry

    lax.fori_loop(0, _NPAIR, step, 0)
    scatter_wait(_NCHUNK - 1, rows1, sem_s1)


def kernel(DNA_x, tf_x, W):
    idx = tf_x.reshape(-1).astype(jnp.int32).reshape(_NW, _IDX_ROWS, _IW)
    emb = _emb_gather(W, idx)
    return (DNA_x, emb.reshape(_B, _L, _D))


# speedup vs baseline: 1.7696x; 1.0002x over previous
import functools

import jax
import jax.numpy as jnp
from jax import lax
from jax.experimental import pallas as pl
from jax.experimental.pallas import tpu as pltpu
from jax.experimental.pallas import tpu_sc as plsc

_NC = 2
_NS = 16
_NW = _NC * _NS

_B = 16384
_L = 39
_D = 32
_N = _B * _L
_BPW = _N // _NW
_CHUNK = 1536
_NCHUNK = _BPW // _CHUNK
_NPAIR = 6


@functools.partial(
    pl.kernel,
    out_type=jax.ShapeDtypeStruct((_N, _D), jnp.float32),
    mesh=plsc.VectorSubcoreMesh(
        core_axis_name="c", subcore_axis_name="s",
        num_cores=_NC, num_subcores=_NS,
    ),
    scratch_types=[
        pltpu.VMEM((_BPW,), jnp.int32),
        pltpu.VMEM((_CHUNK, _D), jnp.float32),
        pltpu.VMEM((_CHUNK, _D), jnp.float32),
        pltpu.SemaphoreType.DMA,
        pltpu.SemaphoreType.DMA,
        pltpu.SemaphoreType.DMA,
        pltpu.SemaphoreType.DMA,
    ],
    compiler_params=pltpu.CompilerParams(use_tc_tiling_on_sc=False),
)
def _emb_gather(table_hbm, idx_hbm, out_hbm, idx_v, rows0, rows1,
                sem_g0, sem_g1, sem_s0, sem_s1):
    wid = lax.axis_index("s") * _NC + lax.axis_index("c")
    pltpu.sync_copy(idx_hbm.at[pl.ds(wid * _BPW, _BPW)], idx_v)
    base = wid * _BPW

    def gather_fire(i, buf, sem):
        pltpu.async_copy(
            table_hbm.at[idx_v.at[pl.ds(i * _CHUNK, _CHUNK)]], buf, sem)

    def gather_wait(i, buf, sem):
        pltpu.make_async_copy(
            table_hbm.at[idx_v.at[pl.ds(i * _CHUNK, _CHUNK)]], buf, sem).wait()

    def scatter_fire(i, buf, sem):
        pltpu.async_copy(buf, out_hbm.at[pl.ds(base + i * _CHUNK, _CHUNK)], sem)

    def scatter_wait(i, buf, sem):
        pltpu.make_async_copy(
            buf, out_hbm.at[pl.ds(base + i * _CHUNK, _CHUNK)], sem).wait()

    gather_fire(0, rows0, sem_g0)

    def step(k, carry):
        i0 = 2 * k
        i1 = i0 + 1

        @pl.when(k > 0)
        def _():
            scatter_wait(i1 - 2, rows1, sem_s1)

        gather_fire(i1, rows1, sem_g1)
        gather_wait(i0, rows0, sem_g0)
        scatter_fire(i0, rows0, sem_s0)
        scatter_wait(i0, rows0, sem_s0)
        gather_fire(i0 + 2, rows0, sem_g0)
        gather_wait(i1, rows1, sem_g1)
        scatter_fire(i1, rows1, sem_s1)
        return carry

    lax.fori_loop(0, _NPAIR, step, 0)
    scatter_wait(_NCHUNK - 2, rows1, sem_s1)
    gather_wait(_NCHUNK - 1, rows0, sem_g0)
    scatter_fire(_NCHUNK - 1, rows0, sem_s0)
    scatter_wait(_NCHUNK - 1, rows0, sem_s0)


def kernel(DNA_x, tf_x, W):
    idx = tf_x.reshape(-1).astype(jnp.int32)
    emb = _emb_gather(W, idx)
    return (DNA_x, emb.reshape(_B, _L, _D))
